# Initial kernel scaffold; baseline (speedup 1.0000x reference)
import jax
import jax.numpy as jnp
from jax.experimental import pallas as pl


def kernel(source, target):
    raise NotImplementedError("write your pallas kernel here")


# speedup vs baseline: 754.9864x; 754.9864x over previous
import functools

import jax
import jax.numpy as jnp
from jax.experimental import pallas as pl
from jax.experimental.pallas import tpu as pltpu

_KERNEL_MUL = 2.0
_KERNEL_NUM = 5


def _mkmmd_kernel(src_ref, tgt_ref, out_ref):
    src = src_ref[:]
    tgt = tgt_ref[:]
    b, d = src.shape
    n = 2 * b

    s1 = jnp.sum(src, axis=0, keepdims=True) + jnp.sum(tgt, axis=0, keepdims=True)
    s2 = (jnp.sum(src * src, axis=0, keepdims=True)
          + jnp.sum(tgt * tgt, axis=0, keepdims=True))
    colsum = (2.0 * n) * s2 - 2.0 * s1 * s1
    w = (d - jax.lax.broadcasted_iota(jnp.int32, (1, d), 1)).astype(jnp.float32)
    bw_sum = jnp.sum(w * colsum)
    bw = bw_sum / (n * n - n) / (_KERNEL_MUL ** (_KERNEL_NUM // 2))
    neg_inv = -1.0 / (bw * (_KERNEL_MUL ** (_KERNEL_NUM - 1)))

    rs = jnp.concatenate([src[1:], src[:1]], axis=0)
    rt = jnp.concatenate([tgt[1:], tgt[:1]], axis=0)

    row = jax.lax.broadcasted_iota(jnp.int32, (d, d), 0)
    col = jax.lax.broadcasted_iota(jnp.int32, (d, d), 1)
    tri = jnp.where(row <= col, 1.0, 0.0).astype(jnp.float32)

    def kset(diff):
        sq = diff * diff
        c = jnp.dot(sq, tri, preferred_element_type=jnp.float32,
                    precision=jax.lax.Precision.HIGHEST)
        z = jnp.exp(c * neg_inv)
        z2 = z * z
        z4 = z2 * z2
        z8 = z4 * z4
        z16 = z8 * z8
        return z + z2 + z4 + z8 + z16

    comb = (kset(src - rs) + kset(tgt - rt)
            - kset(src - rt) - kset(rs - tgt))
    total = jnp.sum(comb, axis=(0, 1), keepdims=True)
    out_ref[:, :] = total * (1.0 / (b * d))


@jax.jit
def kernel(source, target):
    out = pl.pallas_call(
        _mkmmd_kernel,
        out_shape=jax.ShapeDtypeStruct((1, 1), jnp.float32),
        in_specs=[
            pl.BlockSpec(memory_space=pltpu.VMEM),
            pl.BlockSpec(memory_space=pltpu.VMEM),
        ],
        out_specs=pl.BlockSpec(memory_space=pltpu.VMEM),
    )(source, target)
    return out[0, 0]


# speedup vs baseline: 916.3072x; 1.2137x over previous
import functools

import jax
import jax.numpy as jnp
from jax.experimental import pallas as pl
from jax.experimental.pallas import tpu as pltpu

_KERNEL_MUL = 2.0
_KERNEL_NUM = 5


def _mkmmd_kernel(src_ref, tgt_ref, out_ref):
    src = src_ref[:]
    tgt = tgt_ref[:]
    b, d = src.shape
    n = 2 * b

    s1 = jnp.sum(src, axis=0, keepdims=True) + jnp.sum(tgt, axis=0, keepdims=True)
    s2 = (jnp.sum(src * src, axis=0, keepdims=True)
          + jnp.sum(tgt * tgt, axis=0, keepdims=True))
    colsum = (2.0 * n) * s2 - 2.0 * s1 * s1
    w = (d - jax.lax.broadcasted_iota(jnp.int32, (1, d), 1)).astype(jnp.float32)
    bw_sum = jnp.sum(w * colsum)
    bw = bw_sum / (n * n - n) / (_KERNEL_MUL ** (_KERNEL_NUM // 2))
    neg_inv = -1.0 / (bw * (_KERNEL_MUL ** (_KERNEL_NUM - 1)))

    rs = jnp.concatenate([src[1:], src[:1]], axis=0)
    rt = jnp.concatenate([tgt[1:], tgt[:1]], axis=0)

    row = jax.lax.broadcasted_iota(jnp.int32, (d, d), 0)
    col = jax.lax.broadcasted_iota(jnp.int32, (d, d), 1)
    tri = jnp.where(row <= col, 1.0, 0.0).astype(jnp.bfloat16)

    def kset(diff):
        sq = diff * diff
        hi = sq.astype(jnp.bfloat16)
        lo = (sq - hi.astype(jnp.float32)).astype(jnp.bfloat16)
        c = (jnp.dot(hi, tri, preferred_element_type=jnp.float32)
             + jnp.dot(lo, tri, preferred_element_type=jnp.float32))
        z = jnp.exp(c * neg_inv)
        z2 = z * z
        z4 = z2 * z2
        z8 = z4 * z4
        z16 = z8 * z8
        return z + z2 + z4 + z8 + z16

    comb = (kset(src - rs) + kset(tgt - rt)
            - kset(src - rt) - kset(rs - tgt))
    total = jnp.sum(comb, axis=(0, 1), keepdims=True)
    out_ref[:, :] = total * (1.0 / (b * d))


@jax.jit
def kernel(source, target):
    out = pl.pallas_call(
        _mkmmd_kernel,
        out_shape=jax.ShapeDtypeStruct((1, 1), jnp.float32),
        in_specs=[
            pl.BlockSpec(memory_space=pltpu.VMEM),
            pl.BlockSpec(memory_space=pltpu.VMEM),
        ],
        out_specs=pl.BlockSpec(memory_space=pltpu.VMEM),
    )(source, target)
    return out[0, 0]
